# baseline (speedup 1.0000x reference)
import functools

import jax
import jax.numpy as jnp
from jax import lax
from jax.experimental import pallas as pl
from jax.experimental.pallas import tpu as pltpu
from jax.experimental.pallas import tpu_sc as plsc

_N = 10000
_E = 320000
_H = 128
_G = 64
_OUT = 2

_NC = 2
_NS = 16
_NW = _NC * _NS

_NPAD = 10240
_BN = 256
_NB = _NPAD // _BN

_K = 128
_EPT_REAL = _E // _NW
_CHUNKS = 80
_EPT = _CHUNKS * _K
_DUMMY = _NPAD - 1
_STRIPE = _NPAD // _NS

_HI = lax.Precision.HIGHEST


def _deg_body(dst_hbm, ones_hbm, zeros_hbm, out_hbm, idx_v, ones_v, deg_sp):
    c = lax.axis_index("c")
    s = lax.axis_index("s")
    w = c * _NS + s
    pltpu.sync_copy(ones_hbm, ones_v)
    pltpu.sync_copy(zeros_hbm, deg_sp.at[pl.ds(s * _STRIPE, _STRIPE)])
    plsc.subcore_barrier()

    def chunk(i, carry):
        pltpu.sync_copy(dst_hbm.at[w, i], idx_v)
        pltpu.sync_copy(ones_v, deg_sp.at[idx_v], add=True)
        return carry

    lax.fori_loop(0, _CHUNKS, chunk, 0)
    plsc.subcore_barrier()
    pltpu.sync_copy(deg_sp.at[pl.ds(s * _STRIPE, _STRIPE)],
                    out_hbm.at[c, pl.ds(s * _STRIPE, _STRIPE)])


@functools.lru_cache(maxsize=None)
def _deg_call_factory():
    mesh = plsc.VectorSubcoreMesh(core_axis_name="c", subcore_axis_name="s",
                                  num_cores=_NC, num_subcores=_NS)
    return pl.kernel(
        _deg_body,
        out_type=jax.ShapeDtypeStruct((_NC, _NPAD, 16), jnp.float32),
        mesh=mesh,
        scratch_types=[
            pltpu.VMEM((_K,), jnp.int32),
            pltpu.VMEM((_K, 16), jnp.float32),
            pltpu.VMEM_SHARED((_NPAD, 16), jnp.float32),
        ],
    )


def _deg_call(*args):
    return _deg_call_factory()(*args)


def _agg_body(y_hbm, src_hbm, dst_hbm, zeros_hbm, out_hbm,
              sidx, didx, rows0, rows1, acc, semr0, semr1, semi0, semi1):
    c = lax.axis_index("c")
    s = lax.axis_index("s")
    w = c * _NS + s
    pltpu.sync_copy(zeros_hbm, acc.at[pl.ds(s * _STRIPE, _STRIPE)])
    plsc.subcore_barrier()

    rows = (rows0, rows1)
    semr = (semr0, semr1)
    semi = (semi0, semi1)
    pltpu.sync_copy(src_hbm.at[w, 0], sidx.at[0])
    pltpu.sync_copy(dst_hbm.at[w, 0], didx.at[0])
    pltpu.async_copy(src_hbm.at[w, 1], sidx.at[1], semi1)
    pltpu.async_copy(dst_hbm.at[w, 1], didx.at[1], semi1)
    pltpu.async_copy(y_hbm.at[sidx.at[0]], rows0, semr0)

    def step(j, carry):
        for b in range(2):
            i = 2 * j + b
            b1 = 1 - b
            pltpu.make_async_copy(y_hbm.at[sidx.at[b]], rows[b],
                                  semr[b]).wait()

            @pl.when(i + 1 < _CHUNKS)
            def _():
                pltpu.make_async_copy(src_hbm.at[w, 0], sidx.at[b1],
                                      semi[b1]).wait()
                pltpu.make_async_copy(dst_hbm.at[w, 0], didx.at[b1],
                                      semi[b1]).wait()
                pltpu.async_copy(y_hbm.at[sidx.at[b1]], rows[b1], semr[b1])

            pltpu.sync_copy(rows[b], acc.at[didx.at[b]], add=True)

            @pl.when(i + 2 < _CHUNKS)
            def _():
                pltpu.async_copy(src_hbm.at[w, i + 2], sidx.at[b], semi[b])
                pltpu.async_copy(dst_hbm.at[w, i + 2], didx.at[b], semi[b])
        return carry

    lax.fori_loop(0, _CHUNKS // 2, step, 0)
    plsc.subcore_barrier()
    pltpu.sync_copy(acc.at[pl.ds(s * _STRIPE, _STRIPE)],
                    out_hbm.at[c, pl.ds(s * _STRIPE, _STRIPE)])


@functools.lru_cache(maxsize=None)
def _agg_call_factory():
    mesh = plsc.VectorSubcoreMesh(core_axis_name="c", subcore_axis_name="s",
                                  num_cores=_NC, num_subcores=_NS)
    return pl.kernel(
        _agg_body,
        out_type=jax.ShapeDtypeStruct((_NC, _NPAD, _H), jnp.float32),
        mesh=mesh,
        scratch_types=[
            pltpu.VMEM((2, _K), jnp.int32),
            pltpu.VMEM((2, _K), jnp.int32),
            pltpu.VMEM((_K, _H), jnp.float32),
            pltpu.VMEM((_K, _H), jnp.float32),
            pltpu.VMEM_SHARED((_NPAD, _H), jnp.float32),
            pltpu.SemaphoreType.DMA,
            pltpu.SemaphoreType.DMA,
            pltpu.SemaphoreType.DMA,
            pltpu.SemaphoreType.DMA,
        ],
    )


def _agg_call(*args):
    return _agg_call_factory()(*args)


def _onehot(bvec):
    return (bvec[:, None]
            == lax.broadcasted_iota(jnp.int32, (_BN, _G), 1)).astype(jnp.float32)


def _init_body(x_ref, w1_ref, deg_ref, batch_ref, y_ref, dinv_ref, cnt_ref,
               cnt_acc):
    i = pl.program_id(0)
    dinv = lax.rsqrt(1.0 + deg_ref[0] + deg_ref[1])
    dinv_ref[...] = dinv
    y_ref[...] = dinv[:, 0:1] * jnp.dot(x_ref[...], w1_ref[...], precision=_HI)
    S = _onehot(batch_ref[0, 0])
    ones = jnp.ones((_BN, _H), jnp.float32)

    @pl.when(i == 0)
    def _():
        cnt_acc[...] = jnp.zeros_like(cnt_acc)

    cnt_acc[...] += lax.dot_general(S, ones, (((0,), (0,)), ((), ())),
                                    precision=_HI)
    cnt_ref[...] = jnp.maximum(cnt_acc[...], 1.0)


def _tc_init(x_pad, W1, deg, batch3d):
    return pl.pallas_call(
        _init_body,
        grid=(_NB,),
        in_specs=[
            pl.BlockSpec((_BN, _H), lambda i: (i, 0)),
            pl.BlockSpec((_H, _H), lambda i: (0, 0)),
            pl.BlockSpec((_NC, _BN, 16), lambda i: (0, i, 0)),
            pl.BlockSpec((1, 1, _BN), lambda i: (i, 0, 0)),
        ],
        out_specs=[
            pl.BlockSpec((_BN, _H), lambda i: (i, 0)),
            pl.BlockSpec((_BN, 16), lambda i: (i, 0)),
            pl.BlockSpec((_G, _H), lambda i: (0, 0)),
        ],
        out_shape=[
            jax.ShapeDtypeStruct((_NPAD, _H), jnp.float32),
            jax.ShapeDtypeStruct((_NPAD, 16), jnp.float32),
            jax.ShapeDtypeStruct((_G, _H), jnp.float32),
        ],
        scratch_shapes=[pltpu.VMEM((_G, _H), jnp.float32)],
    )(x_pad, W1, deg, batch3d)


def _a_body(p_ref, y_ref, dinv_ref, batch_ref, cnt_ref, b_ref, gms_ref,
            z_ref, mean_ref, stdinv_ref, gsum, sqsum):
    i = pl.program_id(0)
    z = (dinv_ref[:, 0:1] * (p_ref[0] + p_ref[1] + y_ref[...])) + b_ref[...]
    z_ref[...] = z
    S = _onehot(batch_ref[0, 0])

    @pl.when(i == 0)
    def _():
        gsum[...] = jnp.zeros_like(gsum)
        sqsum[...] = jnp.zeros_like(sqsum)

    gsum[...] += lax.dot_general(S, z, (((0,), (0,)), ((), ())), precision=_HI)
    sqsum[...] += lax.dot_general(S, z * z, (((0,), (0,)), ((), ())),
                                  precision=_HI)
    mean = gsum[...] / cnt_ref[...]
    mean_ref[...] = mean
    msm = gms_ref[...] * mean
    var = sqsum[...] / cnt_ref[...] - 2.0 * msm * mean + msm * msm
    stdinv_ref[...] = lax.rsqrt(var + 1e-5)


def _tc_a(p, y, dinv, batch3d, cnt, b, gms):
    return pl.pallas_call(
        _a_body,
        grid=(_NB,),
        in_specs=[
            pl.BlockSpec((_NC, _BN, _H), lambda i: (0, i, 0)),
            pl.BlockSpec((_BN, _H), lambda i: (i, 0)),
            pl.BlockSpec((_BN, 16), lambda i: (i, 0)),
            pl.BlockSpec((1, 1, _BN), lambda i: (i, 0, 0)),
            pl.BlockSpec((_G, _H), lambda i: (0, 0)),
            pl.BlockSpec((1, _H), lambda i: (0, 0)),
            pl.BlockSpec((1, _H), lambda i: (0, 0)),
        ],
        out_specs=[
            pl.BlockSpec((_BN, _H), lambda i: (i, 0)),
            pl.BlockSpec((_G, _H), lambda i: (0, 0)),
            pl.BlockSpec((_G, _H), lambda i: (0, 0)),
        ],
        out_shape=[
            jax.ShapeDtypeStruct((_NPAD, _H), jnp.float32),
            jax.ShapeDtypeStruct((_G, _H), jnp.float32),
            jax.ShapeDtypeStruct((_G, _H), jnp.float32),
        ],
        scratch_shapes=[pltpu.VMEM((_G, _H), jnp.float32),
                        pltpu.VMEM((_G, _H), jnp.float32)],
    )(p, y, dinv, batch3d, cnt, b, gms)


def _b_mid_body(z_ref, mean_ref, stdinv_ref, batch_ref, dinv_ref,
                gw_ref, gb_ref, gms_ref, wn_ref, y_ref):
    i = pl.program_id(0)
    S = _onehot(batch_ref[0, 0])
    sub = z_ref[...] - gms_ref[...] * jnp.dot(S, mean_ref[...], precision=_HI)
    o = gw_ref[...] * sub * jnp.dot(S, stdinv_ref[...], precision=_HI) \
        + gb_ref[...]
    o = jnp.maximum(o, 0.0)
    yn = dinv_ref[:, 0:1] * jnp.dot(o, wn_ref[...], precision=_HI)
    row = lax.broadcasted_iota(jnp.int32, (_BN, _H), 0) + i * _BN
    y_ref[...] = jnp.where(row < _N, yn, 0.0)


def _tc_b_mid(z, mean, stdinv, batch3d, dinv, gw, gb, gms, wn):
    return pl.pallas_call(
        _b_mid_body,
        grid=(_NB,),
        in_specs=[
            pl.BlockSpec((_BN, _H), lambda i: (i, 0)),
            pl.BlockSpec((_G, _H), lambda i: (0, 0)),
            pl.BlockSpec((_G, _H), lambda i: (0, 0)),
            pl.BlockSpec((1, 1, _BN), lambda i: (i, 0, 0)),
            pl.BlockSpec((_BN, 16), lambda i: (i, 0)),
            pl.BlockSpec((1, _H), lambda i: (0, 0)),
            pl.BlockSpec((1, _H), lambda i: (0, 0)),
            pl.BlockSpec((1, _H), lambda i: (0, 0)),
            pl.BlockSpec((_H, _H), lambda i: (0, 0)),
        ],
        out_specs=pl.BlockSpec((_BN, _H), lambda i: (i, 0)),
        out_shape=jax.ShapeDtypeStruct((_NPAD, _H), jnp.float32),
    )(z, mean, stdinv, batch3d, dinv, gw, gb, gms, wn)


def _b_last_body(z_ref, mean_ref, stdinv_ref, batch_ref, cnt_ref,
                 gw_ref, gb_ref, gms_ref,
                 lw1_ref, lb1_ref, lw2_ref, lb2_ref, lw3_ref, lb3_ref,
                 out_ref, psum):
    i = pl.program_id(0)
    S = _onehot(batch_ref[0, 0])
    sub = z_ref[...] - gms_ref[...] * jnp.dot(S, mean_ref[...], precision=_HI)
    o = gw_ref[...] * sub * jnp.dot(S, stdinv_ref[...], precision=_HI) \
        + gb_ref[...]

    @pl.when(i == 0)
    def _():
        psum[...] = jnp.zeros_like(psum)

    psum[...] += lax.dot_general(S, o, (((0,), (0,)), ((), ())), precision=_HI)
    pooled = psum[...] / cnt_ref[...]
    t = jnp.dot(pooled, lw1_ref[...], precision=_HI) + lb1_ref[...]
    t = jnp.dot(t, lw2_ref[...], precision=_HI) + lb2_ref[...]
    out_ref[...] = jnp.dot(t, lw3_ref[...], precision=_HI) + lb3_ref[...]


def _tc_b_last(z, mean, stdinv, batch3d, cnt, gw, gb, gms, lw1, lb1, lw2, lb2,
               lw3p, lb3p):
    small = lambda i: (0, 0)
    return pl.pallas_call(
        _b_last_body,
        grid=(_NB,),
        in_specs=[
            pl.BlockSpec((_BN, _H), lambda i: (i, 0)),
            pl.BlockSpec((_G, _H), small),
            pl.BlockSpec((_G, _H), small),
            pl.BlockSpec((1, 1, _BN), lambda i: (i, 0, 0)),
            pl.BlockSpec((_G, _H), small),
            pl.BlockSpec((1, _H), small),
            pl.BlockSpec((1, _H), small),
            pl.BlockSpec((1, _H), small),
            pl.BlockSpec((_H, _H), small),
            pl.BlockSpec((1, _H), small),
            pl.BlockSpec((_H, _H), small),
            pl.BlockSpec((1, _H), small),
            pl.BlockSpec((_H, _H), small),
            pl.BlockSpec((1, _H), small),
        ],
        out_specs=pl.BlockSpec((_G, _H), small),
        out_shape=jax.ShapeDtypeStruct((_G, _H), jnp.float32),
        scratch_shapes=[pltpu.VMEM((_G, _H), jnp.float32)],
    )(z, mean, stdinv, batch3d, cnt, gw, gb, gms,
      lw1, lb1, lw2, lb2, lw3p, lb3p)


def kernel(x, edge_index, batch, W1, b1, W2, b2, W3, b3,
           gn1_w, gn1_b, gn1_ms, gn2_w, gn2_b, gn2_ms, gn3_w, gn3_b, gn3_ms,
           lw1, lb1, lw2, lb2, lw3, lb3):
    src = edge_index[0].astype(jnp.int32).reshape(_NW, _EPT_REAL)
    dst = edge_index[1].astype(jnp.int32).reshape(_NW, _EPT_REAL)
    epad = jnp.full((_NW, _EPT - _EPT_REAL), _DUMMY, jnp.int32)
    src3 = jnp.concatenate([src, epad], axis=1).reshape(_NW, _CHUNKS, _K)
    dst3 = jnp.concatenate([dst, epad], axis=1).reshape(_NW, _CHUNKS, _K)

    x_pad = jnp.pad(x, ((0, _NPAD - _N), (0, 0)))
    batch3d = jnp.pad(batch.astype(jnp.int32), (0, _NPAD - _N),
                      constant_values=_G).reshape(_NB, 1, _BN)

    ones16 = jnp.ones((_K, 16), jnp.float32)
    zeros16 = jnp.zeros((_STRIPE, 16), jnp.float32)
    zrows = jnp.zeros((_STRIPE, _H), jnp.float32)

    b1r = b1.reshape(1, _H)
    b2r = b2.reshape(1, _H)
    b3r = b3.reshape(1, _H)
    g1w, g1b, g1m = gn1_w.reshape(1, _H), gn1_b.reshape(1, _H), gn1_ms.reshape(1, _H)
    g2w, g2b, g2m = gn2_w.reshape(1, _H), gn2_b.reshape(1, _H), gn2_ms.reshape(1, _H)
    g3w, g3b, g3m = gn3_w.reshape(1, _H), gn3_b.reshape(1, _H), gn3_ms.reshape(1, _H)
    lb1r = lb1.reshape(1, _H)
    lb2r = lb2.reshape(1, _H)
    lw3p = jnp.pad(lw3, ((0, 0), (0, _H - _OUT)))
    lb3p = jnp.pad(lb3, (0, _H - _OUT)).reshape(1, _H)

    deg = _deg_call(dst3, ones16, zeros16)
    y1, dinv, cnt = _tc_init(x_pad, W1, deg, batch3d)

    p = _agg_call(y1, src3, dst3, zrows)
    z, mean, stdinv = _tc_a(p, y1, dinv, batch3d, cnt, b1r, g1m)
    y2 = _tc_b_mid(z, mean, stdinv, batch3d, dinv, g1w, g1b, g1m, W2)

    p = _agg_call(y2, src3, dst3, zrows)
    z, mean, stdinv = _tc_a(p, y2, dinv, batch3d, cnt, b2r, g2m)
    y3 = _tc_b_mid(z, mean, stdinv, batch3d, dinv, g2w, g2b, g2m, W3)

    p = _agg_call(y3, src3, dst3, zrows)
    z, mean, stdinv = _tc_a(p, y3, dinv, batch3d, cnt, b3r, g3m)
    out = _tc_b_last(z, mean, stdinv, batch3d, cnt, g3w, g3b, g3m,
                     lw1, lb1r, lw2, lb2r, lw3p, lb3p)
    return out[:, :_OUT]


# speedup vs baseline: 1.0966x; 1.0966x over previous
import functools

import jax
import jax.numpy as jnp
from jax import lax
from jax.experimental import pallas as pl
from jax.experimental.pallas import tpu as pltpu
from jax.experimental.pallas import tpu_sc as plsc

_N = 10000
_E = 320000
_H = 128
_G = 64
_OUT = 2

_NC = 2
_NS = 16
_NW = _NC * _NS

_NPAD = 10240
_BN = 256
_NB = _NPAD // _BN

_K = 128
_EPT_REAL = _E // _NW
_CHUNKS = -(-_EPT_REAL // _K)
_EPT = _CHUNKS * _K
_DUMMY = _NPAD - 1
_STRIPE = _NPAD // _NS

_HI = lax.Precision.HIGHEST


def _deg_body(dst_hbm, ones_hbm, zeros_hbm, out_hbm, idx_v, ones_v, deg_sp):
    c = lax.axis_index("c")
    s = lax.axis_index("s")
    w = c * _NS + s
    pltpu.sync_copy(ones_hbm, ones_v)
    pltpu.sync_copy(zeros_hbm, deg_sp.at[pl.ds(s * _STRIPE, _STRIPE)])
    plsc.subcore_barrier()

    def chunk(i, carry):
        pltpu.sync_copy(dst_hbm.at[w, i], idx_v)
        pltpu.sync_copy(ones_v, deg_sp.at[idx_v], add=True)
        return carry

    lax.fori_loop(0, _CHUNKS, chunk, 0)
    plsc.subcore_barrier()
    pltpu.sync_copy(deg_sp.at[pl.ds(s * _STRIPE, _STRIPE)],
                    out_hbm.at[c, pl.ds(s * _STRIPE, _STRIPE)])


@functools.lru_cache(maxsize=None)
def _deg_call_factory():
    mesh = plsc.VectorSubcoreMesh(core_axis_name="c", subcore_axis_name="s",
                                  num_cores=_NC, num_subcores=_NS)
    return pl.kernel(
        _deg_body,
        out_type=jax.ShapeDtypeStruct((_NC, _NPAD, 16), jnp.float32),
        mesh=mesh,
        scratch_types=[
            pltpu.VMEM((_K,), jnp.int32),
            pltpu.VMEM((_K, 16), jnp.float32),
            pltpu.VMEM_SHARED((_NPAD, 16), jnp.float32),
        ],
    )


def _deg_call(*args):
    return _deg_call_factory()(*args)


def _agg_body(y_hbm, src_hbm, dst_hbm, zeros_hbm, out_hbm,
              sidx, didx, rows, acc, sem):
    c = lax.axis_index("c")
    s = lax.axis_index("s")
    w = c * _NS + s
    pltpu.sync_copy(zeros_hbm, acc.at[pl.ds(s * _STRIPE, _STRIPE)])
    plsc.subcore_barrier()

    def chunk(i, carry):
        pltpu.sync_copy(src_hbm.at[w, i], sidx)
        pltpu.sync_copy(dst_hbm.at[w, i], didx)
        pltpu.async_copy(y_hbm.at[sidx], rows, sem).wait()
        pltpu.sync_copy(rows, acc.at[didx], add=True)
        return carry

    lax.fori_loop(0, _CHUNKS, chunk, 0)
    plsc.subcore_barrier()
    pltpu.sync_copy(acc.at[pl.ds(s * _STRIPE, _STRIPE)],
                    out_hbm.at[c, pl.ds(s * _STRIPE, _STRIPE)])


@functools.lru_cache(maxsize=None)
def _agg_call_factory():
    mesh = plsc.VectorSubcoreMesh(core_axis_name="c", subcore_axis_name="s",
                                  num_cores=_NC, num_subcores=_NS)
    return pl.kernel(
        _agg_body,
        out_type=jax.ShapeDtypeStruct((_NC, _NPAD, _H), jnp.float32),
        mesh=mesh,
        scratch_types=[
            pltpu.VMEM((_K,), jnp.int32),
            pltpu.VMEM((_K,), jnp.int32),
            pltpu.VMEM((_K, _H), jnp.float32),
            pltpu.VMEM_SHARED((_NPAD, _H), jnp.float32),
            pltpu.SemaphoreType.DMA,
        ],
    )


def _agg_call(*args):
    return _agg_call_factory()(*args)


def _onehot(bvec):
    return (bvec[:, None]
            == lax.broadcasted_iota(jnp.int32, (_BN, _G), 1)).astype(jnp.float32)


def _init_body(x_ref, w1_ref, deg_ref, batch_ref, y_ref, dinv_ref, cnt_ref,
               cnt_acc):
    i = pl.program_id(0)
    dinv = lax.rsqrt(1.0 + deg_ref[0] + deg_ref[1])
    dinv_ref[...] = dinv
    y_ref[...] = dinv[:, 0:1] * jnp.dot(x_ref[...], w1_ref[...], precision=_HI)
    S = _onehot(batch_ref[0, 0])
    ones = jnp.ones((_BN, _H), jnp.float32)

    @pl.when(i == 0)
    def _():
        cnt_acc[...] = jnp.zeros_like(cnt_acc)

    cnt_acc[...] += lax.dot_general(S, ones, (((0,), (0,)), ((), ())),
                                    precision=_HI)
    cnt_ref[...] = jnp.maximum(cnt_acc[...], 1.0)


def _tc_init(x_pad, W1, deg, batch3d):
    return pl.pallas_call(
        _init_body,
        grid=(_NB,),
        in_specs=[
            pl.BlockSpec((_BN, _H), lambda i: (i, 0)),
            pl.BlockSpec((_H, _H), lambda i: (0, 0)),
            pl.BlockSpec((_NC, _BN, 16), lambda i: (0, i, 0)),
            pl.BlockSpec((1, 1, _BN), lambda i: (i, 0, 0)),
        ],
        out_specs=[
            pl.BlockSpec((_BN, _H), lambda i: (i, 0)),
            pl.BlockSpec((_BN, 16), lambda i: (i, 0)),
            pl.BlockSpec((_G, _H), lambda i: (0, 0)),
        ],
        out_shape=[
            jax.ShapeDtypeStruct((_NPAD, _H), jnp.float32),
            jax.ShapeDtypeStruct((_NPAD, 16), jnp.float32),
            jax.ShapeDtypeStruct((_G, _H), jnp.float32),
        ],
        scratch_shapes=[pltpu.VMEM((_G, _H), jnp.float32)],
    )(x_pad, W1, deg, batch3d)


def _a_body(p_ref, y_ref, dinv_ref, batch_ref, cnt_ref, b_ref, gms_ref,
            z_ref, mean_ref, stdinv_ref, gsum, sqsum):
    i = pl.program_id(0)
    z = (dinv_ref[:, 0:1] * (p_ref[0] + p_ref[1] + y_ref[...])) + b_ref[...]
    z_ref[...] = z
    S = _onehot(batch_ref[0, 0])

    @pl.when(i == 0)
    def _():
        gsum[...] = jnp.zeros_like(gsum)
        sqsum[...] = jnp.zeros_like(sqsum)

    gsum[...] += lax.dot_general(S, z, (((0,), (0,)), ((), ())), precision=_HI)
    sqsum[...] += lax.dot_general(S, z * z, (((0,), (0,)), ((), ())),
                                  precision=_HI)
    mean = gsum[...] / cnt_ref[...]
    mean_ref[...] = mean
    msm = gms_ref[...] * mean
    var = sqsum[...] / cnt_ref[...] - 2.0 * msm * mean + msm * msm
    stdinv_ref[...] = lax.rsqrt(var + 1e-5)


def _tc_a(p, y, dinv, batch3d, cnt, b, gms):
    return pl.pallas_call(
        _a_body,
        grid=(_NB,),
        in_specs=[
            pl.BlockSpec((_NC, _BN, _H), lambda i: (0, i, 0)),
            pl.BlockSpec((_BN, _H), lambda i: (i, 0)),
            pl.BlockSpec((_BN, 16), lambda i: (i, 0)),
            pl.BlockSpec((1, 1, _BN), lambda i: (i, 0, 0)),
            pl.BlockSpec((_G, _H), lambda i: (0, 0)),
            pl.BlockSpec((1, _H), lambda i: (0, 0)),
            pl.BlockSpec((1, _H), lambda i: (0, 0)),
        ],
        out_specs=[
            pl.BlockSpec((_BN, _H), lambda i: (i, 0)),
            pl.BlockSpec((_G, _H), lambda i: (0, 0)),
            pl.BlockSpec((_G, _H), lambda i: (0, 0)),
        ],
        out_shape=[
            jax.ShapeDtypeStruct((_NPAD, _H), jnp.float32),
            jax.ShapeDtypeStruct((_G, _H), jnp.float32),
            jax.ShapeDtypeStruct((_G, _H), jnp.float32),
        ],
        scratch_shapes=[pltpu.VMEM((_G, _H), jnp.float32),
                        pltpu.VMEM((_G, _H), jnp.float32)],
    )(p, y, dinv, batch3d, cnt, b, gms)


def _b_mid_body(z_ref, mean_ref, stdinv_ref, batch_ref, dinv_ref,
                gw_ref, gb_ref, gms_ref, wn_ref, y_ref):
    i = pl.program_id(0)
    S = _onehot(batch_ref[0, 0])
    sub = z_ref[...] - gms_ref[...] * jnp.dot(S, mean_ref[...], precision=_HI)
    o = gw_ref[...] * sub * jnp.dot(S, stdinv_ref[...], precision=_HI) \
        + gb_ref[...]
    o = jnp.maximum(o, 0.0)
    yn = dinv_ref[:, 0:1] * jnp.dot(o, wn_ref[...], precision=_HI)
    row = lax.broadcasted_iota(jnp.int32, (_BN, _H), 0) + i * _BN
    y_ref[...] = jnp.where(row < _N, yn, 0.0)


def _tc_b_mid(z, mean, stdinv, batch3d, dinv, gw, gb, gms, wn):
    return pl.pallas_call(
        _b_mid_body,
        grid=(_NB,),
        in_specs=[
            pl.BlockSpec((_BN, _H), lambda i: (i, 0)),
            pl.BlockSpec((_G, _H), lambda i: (0, 0)),
            pl.BlockSpec((_G, _H), lambda i: (0, 0)),
            pl.BlockSpec((1, 1, _BN), lambda i: (i, 0, 0)),
            pl.BlockSpec((_BN, 16), lambda i: (i, 0)),
            pl.BlockSpec((1, _H), lambda i: (0, 0)),
            pl.BlockSpec((1, _H), lambda i: (0, 0)),
            pl.BlockSpec((1, _H), lambda i: (0, 0)),
            pl.BlockSpec((_H, _H), lambda i: (0, 0)),
        ],
        out_specs=pl.BlockSpec((_BN, _H), lambda i: (i, 0)),
        out_shape=jax.ShapeDtypeStruct((_NPAD, _H), jnp.float32),
    )(z, mean, stdinv, batch3d, dinv, gw, gb, gms, wn)


def _b_last_body(z_ref, mean_ref, stdinv_ref, batch_ref, cnt_ref,
                 gw_ref, gb_ref, gms_ref,
                 lw1_ref, lb1_ref, lw2_ref, lb2_ref, lw3_ref, lb3_ref,
                 out_ref, psum):
    i = pl.program_id(0)
    S = _onehot(batch_ref[0, 0])
    sub = z_ref[...] - gms_ref[...] * jnp.dot(S, mean_ref[...], precision=_HI)
    o = gw_ref[...] * sub * jnp.dot(S, stdinv_ref[...], precision=_HI) \
        + gb_ref[...]

    @pl.when(i == 0)
    def _():
        psum[...] = jnp.zeros_like(psum)

    psum[...] += lax.dot_general(S, o, (((0,), (0,)), ((), ())), precision=_HI)
    pooled = psum[...] / cnt_ref[...]
    t = jnp.dot(pooled, lw1_ref[...], precision=_HI) + lb1_ref[...]
    t = jnp.dot(t, lw2_ref[...], precision=_HI) + lb2_ref[...]
    out_ref[...] = jnp.dot(t, lw3_ref[...], precision=_HI) + lb3_ref[...]


def _tc_b_last(z, mean, stdinv, batch3d, cnt, gw, gb, gms, lw1, lb1, lw2, lb2,
               lw3p, lb3p):
    small = lambda i: (0, 0)
    return pl.pallas_call(
        _b_last_body,
        grid=(_NB,),
        in_specs=[
            pl.BlockSpec((_BN, _H), lambda i: (i, 0)),
            pl.BlockSpec((_G, _H), small),
            pl.BlockSpec((_G, _H), small),
            pl.BlockSpec((1, 1, _BN), lambda i: (i, 0, 0)),
            pl.BlockSpec((_G, _H), small),
            pl.BlockSpec((1, _H), small),
            pl.BlockSpec((1, _H), small),
            pl.BlockSpec((1, _H), small),
            pl.BlockSpec((_H, _H), small),
            pl.BlockSpec((1, _H), small),
            pl.BlockSpec((_H, _H), small),
            pl.BlockSpec((1, _H), small),
            pl.BlockSpec((_H, _H), small),
            pl.BlockSpec((1, _H), small),
        ],
        out_specs=pl.BlockSpec((_G, _H), small),
        out_shape=jax.ShapeDtypeStruct((_G, _H), jnp.float32),
        scratch_shapes=[pltpu.VMEM((_G, _H), jnp.float32)],
    )(z, mean, stdinv, batch3d, cnt, gw, gb, gms,
      lw1, lb1, lw2, lb2, lw3p, lb3p)


def kernel(x, edge_index, batch, W1, b1, W2, b2, W3, b3,
           gn1_w, gn1_b, gn1_ms, gn2_w, gn2_b, gn2_ms, gn3_w, gn3_b, gn3_ms,
           lw1, lb1, lw2, lb2, lw3, lb3):
    src = edge_index[0].astype(jnp.int32).reshape(_NW, _EPT_REAL)
    dst = edge_index[1].astype(jnp.int32).reshape(_NW, _EPT_REAL)
    epad = jnp.full((_NW, _EPT - _EPT_REAL), _DUMMY, jnp.int32)
    src3 = jnp.concatenate([src, epad], axis=1).reshape(_NW, _CHUNKS, _K)
    dst3 = jnp.concatenate([dst, epad], axis=1).reshape(_NW, _CHUNKS, _K)

    x_pad = jnp.pad(x, ((0, _NPAD - _N), (0, 0)))
    batch3d = jnp.pad(batch.astype(jnp.int32), (0, _NPAD - _N),
                      constant_values=_G).reshape(_NB, 1, _BN)

    ones16 = jnp.ones((_K, 16), jnp.float32)
    zeros16 = jnp.zeros((_STRIPE, 16), jnp.float32)
    zrows = jnp.zeros((_STRIPE, _H), jnp.float32)

    b1r = b1.reshape(1, _H)
    b2r = b2.reshape(1, _H)
    b3r = b3.reshape(1, _H)
    g1w, g1b, g1m = gn1_w.reshape(1, _H), gn1_b.reshape(1, _H), gn1_ms.reshape(1, _H)
    g2w, g2b, g2m = gn2_w.reshape(1, _H), gn2_b.reshape(1, _H), gn2_ms.reshape(1, _H)
    g3w, g3b, g3m = gn3_w.reshape(1, _H), gn3_b.reshape(1, _H), gn3_ms.reshape(1, _H)
    lb1r = lb1.reshape(1, _H)
    lb2r = lb2.reshape(1, _H)
    lw3p = jnp.pad(lw3, ((0, 0), (0, _H - _OUT)))
    lb3p = jnp.pad(lb3, (0, _H - _OUT)).reshape(1, _H)

    deg = _deg_call(dst3, ones16, zeros16)
    y1, dinv, cnt = _tc_init(x_pad, W1, deg, batch3d)

    p = _agg_call(y1, src3, dst3, zrows)
    z, mean, stdinv = _tc_a(p, y1, dinv, batch3d, cnt, b1r, g1m)
    y2 = _tc_b_mid(z, mean, stdinv, batch3d, dinv, g1w, g1b, g1m, W2)

    p = _agg_call(y2, src3, dst3, zrows)
    z, mean, stdinv = _tc_a(p, y2, dinv, batch3d, cnt, b2r, g2m)
    y3 = _tc_b_mid(z, mean, stdinv, batch3d, dinv, g2w, g2b, g2m, W3)

    p = _agg_call(y3, src3, dst3, zrows)
    z, mean, stdinv = _tc_a(p, y3, dinv, batch3d, cnt, b3r, g3m)
    out = _tc_b_last(z, mean, stdinv, batch3d, cnt, g3w, g3b, g3m,
                     lw1, lb1r, lw2, lb2r, lw3p, lb3p)
    return out[:, :_OUT]
